# baseline (speedup 1.0000x reference)
import numpy as np
import jax
import jax.numpy as jnp
from jax.experimental import pallas as pl

_NUM_SAMPLES = 16
_T = 128
_ROW = 3 * 224 * 224

_IDX = np.clip(np.linspace(0.0, _T - 1, _NUM_SAMPLES), 0, _T - 1).astype(np.int32)


def _copy_kernel(x_ref, o_ref):
    o_ref[...] = x_ref[...]


def kernel(x):
    x2 = x.reshape(_T, 1, _ROW)

    def in_map(i):
        return ((i * (_T - 1)) // (_NUM_SAMPLES - 1), 0, 0)

    out = pl.pallas_call(
        _copy_kernel,
        grid=(_NUM_SAMPLES,),
        in_specs=[pl.BlockSpec((1, 1, _ROW), in_map)],
        out_specs=pl.BlockSpec((1, 1, _ROW), lambda i: (i, 0, 0)),
        out_shape=jax.ShapeDtypeStruct((_NUM_SAMPLES, 1, _ROW), x.dtype),
    )(x2)
    return out.reshape(_NUM_SAMPLES, 3, 224, 224)
